# baseline (speedup 1.0000x reference)
import functools

import jax
import jax.numpy as jnp
from jax import lax
from jax.experimental import pallas as pl
from jax.experimental.pallas import tpu as pltpu
from jax.experimental.pallas import tpu_sc as plsc

B, S, D = 2, 2048, 2048
H = 4096
E = 8
K = 2
N = B * S
M = 256
P = N * K + E * M
T = P // M
C = 512
HBK = 512

NC, NS = 2, 16
NW = NC * NS
TPW = N // NW
CH = 16
NJ = TPW // CH
CH2 = 8
NJ2 = TPW // CH2

_HIGHEST = jax.lax.Precision.HIGHEST


def _router_body(logits_ref, gates_ref, eidx_ref, rank_ref,
                 counts_ref, base_ref):
    i = pl.program_id(0)

    @pl.when(i == 0)
    def _():
        base_ref[...] = jnp.zeros_like(base_ref)

    logits = logits_ref[...]
    m = jnp.max(logits, axis=1, keepdims=True)
    p = jnp.exp(logits - m)
    p = p / jnp.sum(p, axis=1, keepdims=True)

    iota = lax.broadcasted_iota(jnp.int32, (C, E), 1)
    m1 = jnp.max(p, axis=1, keepdims=True)
    i1 = jnp.min(jnp.where(p == m1, iota, E), axis=1, keepdims=True)
    p2 = jnp.where(iota == i1, -1.0, p)
    m2 = jnp.max(p2, axis=1, keepdims=True)
    i2 = jnp.min(jnp.where(p2 == m2, iota, E), axis=1, keepdims=True)

    oh1 = (iota == i1).astype(jnp.float32)
    oh2 = (iota == i2).astype(jnp.float32)
    sel = oh1 + oh2

    r_i = lax.broadcasted_iota(jnp.int32, (C, C), 0)
    c_i = lax.broadcasted_iota(jnp.int32, (C, C), 1)
    ltri = (r_i > c_i).astype(jnp.float32)
    rk = lax.dot_general(ltri, sel, (((1,), (0,)), ((), ())),
                         precision=_HIGHEST,
                         preferred_element_type=jnp.float32)
    rk = rk + base_ref[...]

    gates_ref[...] = jnp.concatenate([m1, m2], axis=1)
    eidx_ref[...] = jnp.concatenate([i1, i2], axis=1)
    r1 = jnp.sum(oh1 * rk, axis=1, keepdims=True)
    r2 = jnp.sum(oh2 * rk, axis=1, keepdims=True)
    rank_ref[...] = jnp.concatenate([r1, r2], axis=1).astype(jnp.int32)

    base_ref[...] = base_ref[...] + jnp.sum(sel, axis=0, keepdims=True)

    @pl.when(i == pl.num_programs(0) - 1)
    def _():
        counts_ref[...] = base_ref[...].astype(jnp.int32)


def _router(logits):
    return pl.pallas_call(
        _router_body,
        grid=(N // C,),
        in_specs=[
            pl.BlockSpec((C, E), lambda i: (i, 0)),
        ],
        out_specs=[
            pl.BlockSpec((C, K), lambda i: (i, 0)),
            pl.BlockSpec((C, K), lambda i: (i, 0)),
            pl.BlockSpec((C, K), lambda i: (i, 0)),
            pl.BlockSpec((1, E), lambda i: (0, 0)),
        ],
        out_shape=[
            jax.ShapeDtypeStruct((N, K), jnp.float32),
            jax.ShapeDtypeStruct((N, K), jnp.int32),
            jax.ShapeDtypeStruct((N, K), jnp.int32),
            jax.ShapeDtypeStruct((1, E), jnp.int32),
        ],
        scratch_shapes=[pltpu.VMEM((1, E), jnp.float32)],
    )(logits)


def _meta_body(counts_ref, eidx_ref, rank_ref, dest_ref, be_ref, tv_ref):
    c = counts_ref[...]
    pc = ((c + (M - 1)) // M) * M
    iota_r = lax.broadcasted_iota(jnp.int32, (E, E), 0)
    iota_c = lax.broadcasted_iota(jnp.int32, (E, E), 1)
    sutri = (iota_r < iota_c).astype(jnp.float32)
    off = lax.dot_general(pc.astype(jnp.float32), sutri,
                          (((1,), (0,)), ((), ())), precision=_HIGHEST,
                          preferred_element_type=jnp.float32).astype(jnp.int32)
    vend = off + c
    pcum = off + pc

    eidx = eidx_ref[...]
    dest = rank_ref[...]
    for e in range(E):
        dest = dest + jnp.where(eidx == e, off[:, e:e + 1], 0)
    dest_ref[...] = dest

    iota8 = lax.broadcasted_iota(jnp.int32, (1, E), 1)
    max_live = jnp.max(jnp.where(c > 0, iota8, 0))
    tt = lax.broadcasted_iota(jnp.int32, (1, T), 1) * M
    be = jnp.zeros((1, T), jnp.int32)
    for e in range(E):
        be = be + (tt >= pcum[:, e:e + 1]).astype(jnp.int32)
    be = jnp.minimum(be, max_live)
    be_ref[...] = be
    vend_t = jnp.zeros((1, T), jnp.int32)
    for e in range(E):
        vend_t = vend_t + jnp.where(be == e, vend[:, e:e + 1], 0)
    tv_ref[...] = jnp.clip(vend_t - tt, 0, M)


def _meta(counts, eidx, rank):
    return pl.pallas_call(
        _meta_body,
        out_shape=[
            jax.ShapeDtypeStruct((N, K), jnp.int32),
            jax.ShapeDtypeStruct((1, T), jnp.int32),
            jax.ShapeDtypeStruct((1, T), jnp.int32),
        ],
    )(counts, eidx, rank)


def _sc_mesh():
    return plsc.VectorSubcoreMesh(core_axis_name="c", subcore_axis_name="s")


def _dispatch_body(x_hbm, dest_hbm, destf_hbm, gatesf_hbm, xd_hbm, gs_hbm,
                   idx_v, idxf_v, gf_v, rows_v, sem_l, sem_s, sem_g):
    wid = lax.axis_index("s") * NC + lax.axis_index("c")
    base = wid * TPW
    pltpu.sync_copy(dest_hbm.at[wid], idx_v)
    pltpu.sync_copy(destf_hbm.at[wid], idxf_v)
    pltpu.sync_copy(gatesf_hbm.at[wid], gf_v)
    gate_sc = [
        pltpu.async_copy(gf_v.at[b], gs_hbm.at[idxf_v.at[b]], sem_g)
        for b in range(K * TPW // 128)
    ]
    loads = {
        j: pltpu.async_copy(x_hbm.at[pl.ds(base + j * CH, CH)],
                            rows_v.at[j % 3], sem_l)
        for j in range(min(2, NJ))
    }
    scat = {}
    waited = set()
    for j in range(NJ):
        loads[j].wait()
        if j + 2 < NJ:
            for h in scat.get(j - 1, ()):
                h.wait()
            waited.add(j - 1)
            loads[j + 2] = pltpu.async_copy(
                x_hbm.at[pl.ds(base + (j + 2) * CH, CH)],
                rows_v.at[(j + 2) % 3], sem_l)
        scat[j] = [
            pltpu.async_copy(rows_v.at[j % 3], xd_hbm.at[idx_v.at[j, k]],
                             sem_s)
            for k in range(K)
        ]
    for j in range(NJ):
        if j not in waited:
            for h in scat[j]:
                h.wait()
    for h in gate_sc:
        h.wait()


def _dispatch_sc(xf, dest3, destf, gatesf):
    return pl.kernel(
        _dispatch_body,
        mesh=_sc_mesh(),
        out_type=[
            jax.ShapeDtypeStruct((P, D), jnp.float32),
            jax.ShapeDtypeStruct((P,), jnp.float32),
        ],
        scratch_types=[
            pltpu.VMEM((NJ, K, CH), jnp.int32),
            pltpu.VMEM((K * TPW // 128, 128), jnp.int32),
            pltpu.VMEM((K * TPW // 128, 128), jnp.float32),
            pltpu.VMEM((3, CH, D), jnp.float32),
            pltpu.SemaphoreType.DMA,
            pltpu.SemaphoreType.DMA,
            pltpu.SemaphoreType.DMA,
        ],
    )(xf, dest3, destf, gatesf)


HB2 = H // 2


def _ffn1_body(be_ref, tv_ref, xd_ref, w1_ref, b1_ref, h_ref):
    t = pl.program_id(0)

    @pl.when(tv_ref[t] > 0)
    def _():
        h = lax.dot_general(xd_ref[...], w1_ref[0], (((1,), (0,)), ((), ())),
                            preferred_element_type=jnp.float32)
        h = h + b1_ref[0]
        h = 0.5 * h * (1.0 + lax.erf(h * 0.7071067811865476))
        h_ref[...] = h.astype(jnp.bfloat16)


def _ffn2_body(be_ref, tv_ref, h_ref, w2_ref, b2_ref, gs_ref, out_ref):
    t = pl.program_id(0)
    hq = pl.program_id(1)

    @pl.when(tv_ref[t] > 0)
    def _():
        part = lax.dot_general(h_ref[...].astype(jnp.float32), w2_ref[0],
                               (((1,), (0,)), ((), ())),
                               preferred_element_type=jnp.float32)

        @pl.when(hq == 0)
        def _():
            out_ref[...] = part

        @pl.when(hq == 1)
        def _():
            g = gs_ref[0, 0, :]
            out_ref[...] = (out_ref[...] + part + b2_ref[0]) * g[:, None]


def _ffn(be_arr, tv_arr, xd, W1, W2, b1, b2, gslot):
    def live(t, tv):
        return jnp.where(tv[t] > 0, t, 0)

    def snake(t, hq):
        return jnp.where(t % 2 == 0, hq, 1 - hq)

    spec1 = pltpu.PrefetchScalarGridSpec(
        num_scalar_prefetch=2,
        grid=(T, 2),
        in_specs=[
            pl.BlockSpec((M, D), lambda t, hq, be, tv: (live(t, tv), 0)),
            pl.BlockSpec((1, D, HB2),
                         lambda t, hq, be, tv: (be[t], 0, snake(t, hq))),
            pl.BlockSpec((1, 1, HB2),
                         lambda t, hq, be, tv: (be[t], 0, snake(t, hq))),
        ],
        out_specs=pl.BlockSpec((M, HB2),
                               lambda t, hq, be, tv: (t, snake(t, hq))),
    )
    h = pl.pallas_call(
        _ffn1_body,
        grid_spec=spec1,
        out_shape=jax.ShapeDtypeStruct((P, H), jnp.bfloat16),
    )(be_arr, tv_arr, xd, W1, b1)
    spec2 = pltpu.PrefetchScalarGridSpec(
        num_scalar_prefetch=2,
        grid=(T, 2),
        in_specs=[
            pl.BlockSpec((M, HB2),
                         lambda t, hq, be, tv: (live(t, tv), snake(t, hq))),
            pl.BlockSpec((1, HB2, D),
                         lambda t, hq, be, tv: (be[t], snake(t, hq), 0)),
            pl.BlockSpec((1, 1, D), lambda t, hq, be, tv: (be[t], 0, 0)),
            pl.BlockSpec((1, 1, M),
                         lambda t, hq, be, tv: (live(t, tv), 0, 0)),
        ],
        out_specs=pl.BlockSpec((M, D), lambda t, hq, be, tv: (t, 0)),
    )
    return pl.pallas_call(
        _ffn2_body,
        grid_spec=spec2,
        out_shape=jax.ShapeDtypeStruct((P, D), jnp.float32),
    )(be_arr, tv_arr, h, W2, b2, gslot)


def _combine_body(out_hbm, dest_hbm, y_hbm, idx_v, r0, r1, sem_g, sem_y):
    wid = lax.axis_index("s") * NC + lax.axis_index("c")
    base = wid * TPW
    pltpu.sync_copy(dest_hbm.at[wid], idx_v)

    def gathers(j, b):
        return [
            pltpu.async_copy(out_hbm.at[idx_v.at[j, 0]], r0.at[b], sem_g),
            pltpu.async_copy(out_hbm.at[idx_v.at[j, 1]], r1.at[b], sem_g),
        ]

    pend = gathers(0, 0)
    ysc = None
    for j in range(NJ2):
        b = j % 2
        for hdl in pend:
            hdl.wait()
        if j + 1 < NJ2:
            if ysc is not None:
                ysc.wait()
                ysc = None
            pend = gathers(j + 1, 1 - b)

        def body_i(i, _):
            def body_c(cc, _):
                a = r0[b, i, pl.ds(cc * 16, 16)]
                bb = r1[b, i, pl.ds(cc * 16, 16)]
                r0[b, i, pl.ds(cc * 16, 16)] = a + bb
                return 0

            lax.fori_loop(0, D // 16, body_c, 0)
            return 0

        lax.fori_loop(0, CH2, body_i, 0)
        if ysc is not None:
            ysc.wait()
        ysc = pltpu.async_copy(r0.at[b], y_hbm.at[pl.ds(base + j * CH2, CH2)],
                               sem_y)
    ysc.wait()


def _combine_sc(out, dest2):
    return pl.kernel(
        _combine_body,
        mesh=_sc_mesh(),
        out_type=jax.ShapeDtypeStruct((N, D), jnp.float32),
        scratch_types=[
            pltpu.VMEM((NJ2, K, CH2), jnp.int32),
            pltpu.VMEM((2, CH2, D), jnp.float32),
            pltpu.VMEM((2, CH2, D), jnp.float32),
            pltpu.SemaphoreType.DMA,
            pltpu.SemaphoreType.DMA,
        ],
    )(out, dest2)


@jax.jit
def kernel(x, Wr, br, W1, b1, W2, b2):
    xf = x.reshape(N, D)
    logits = xf @ Wr + br
    gates, eidx, rank, counts = _router(logits)
    dest, be, tv = _meta(counts, eidx, rank)
    dest3 = dest.reshape(NW, NJ, CH, K).transpose(0, 1, 3, 2)
    gates3 = gates.reshape(NW, NJ, CH, K).transpose(0, 1, 3, 2)
    destf = dest3.reshape(NW, K * TPW // 128, 128)
    gatesf = gates3.reshape(NW, K * TPW // 128, 128)
    dest2 = dest.reshape(NW, NJ2, CH2, K).transpose(0, 1, 3, 2)
    xd, gslot = _dispatch_sc(xf, dest3, destf, gatesf)
    out = _ffn(be.reshape(T), tv.reshape(T), xd, W1, W2,
               b1.reshape(E, 1, H), b2.reshape(E, 1, D),
               gslot.reshape(T, 1, M))
    y = _combine_sc(out, dest2)
    return y.astype(jnp.float32).reshape(B, S, D)


# speedup vs baseline: 1.1352x; 1.1352x over previous
import functools

import jax
import jax.numpy as jnp
from jax import lax
from jax.experimental import pallas as pl
from jax.experimental.pallas import tpu as pltpu
from jax.experimental.pallas import tpu_sc as plsc

B, S, D = 2, 2048, 2048
H = 4096
E = 8
K = 2
N = B * S
M = 256
P = N * K + E * M
T = P // M
C = 512
HBK = 512

NC, NS = 2, 16
NW = NC * NS
TPW = N // NW
CH = 16
NJ = TPW // CH
CH2 = 8
NJ2 = TPW // CH2

_HIGHEST = jax.lax.Precision.HIGHEST


def _router_body(logits_ref, gates_ref, eidx_ref, rank_ref,
                 counts_ref, base_ref):
    i = pl.program_id(0)

    @pl.when(i == 0)
    def _():
        base_ref[...] = jnp.zeros_like(base_ref)

    logits = logits_ref[...]
    m = jnp.max(logits, axis=1, keepdims=True)
    p = jnp.exp(logits - m)
    p = p / jnp.sum(p, axis=1, keepdims=True)

    iota = lax.broadcasted_iota(jnp.int32, (C, E), 1)
    m1 = jnp.max(p, axis=1, keepdims=True)
    i1 = jnp.min(jnp.where(p == m1, iota, E), axis=1, keepdims=True)
    p2 = jnp.where(iota == i1, -1.0, p)
    m2 = jnp.max(p2, axis=1, keepdims=True)
    i2 = jnp.min(jnp.where(p2 == m2, iota, E), axis=1, keepdims=True)

    oh1 = (iota == i1).astype(jnp.float32)
    oh2 = (iota == i2).astype(jnp.float32)
    sel = oh1 + oh2

    r_i = lax.broadcasted_iota(jnp.int32, (C, C), 0)
    c_i = lax.broadcasted_iota(jnp.int32, (C, C), 1)
    ltri = (r_i > c_i).astype(jnp.float32)
    rk = lax.dot_general(ltri, sel, (((1,), (0,)), ((), ())),
                         precision=_HIGHEST,
                         preferred_element_type=jnp.float32)
    rk = rk + base_ref[...]

    gates_ref[...] = jnp.concatenate([m1, m2], axis=1)
    eidx_ref[...] = jnp.concatenate([i1, i2], axis=1)
    r1 = jnp.sum(oh1 * rk, axis=1, keepdims=True)
    r2 = jnp.sum(oh2 * rk, axis=1, keepdims=True)
    rank_ref[...] = jnp.concatenate([r1, r2], axis=1).astype(jnp.int32)

    base_ref[...] = base_ref[...] + jnp.sum(sel, axis=0, keepdims=True)

    @pl.when(i == pl.num_programs(0) - 1)
    def _():
        counts_ref[...] = base_ref[...].astype(jnp.int32)


def _router(logits):
    return pl.pallas_call(
        _router_body,
        grid=(N // C,),
        in_specs=[
            pl.BlockSpec((C, E), lambda i: (i, 0)),
        ],
        out_specs=[
            pl.BlockSpec((C, K), lambda i: (i, 0)),
            pl.BlockSpec((C, K), lambda i: (i, 0)),
            pl.BlockSpec((C, K), lambda i: (i, 0)),
            pl.BlockSpec((1, E), lambda i: (0, 0)),
        ],
        out_shape=[
            jax.ShapeDtypeStruct((N, K), jnp.float32),
            jax.ShapeDtypeStruct((N, K), jnp.int32),
            jax.ShapeDtypeStruct((N, K), jnp.int32),
            jax.ShapeDtypeStruct((1, E), jnp.int32),
        ],
        scratch_shapes=[pltpu.VMEM((1, E), jnp.float32)],
    )(logits)


def _meta_body(counts_ref, eidx_ref, rank_ref, dest_ref, be_ref, tv_ref):
    c = counts_ref[...]
    pc = ((c + (M - 1)) // M) * M
    iota_r = lax.broadcasted_iota(jnp.int32, (E, E), 0)
    iota_c = lax.broadcasted_iota(jnp.int32, (E, E), 1)
    sutri = (iota_r < iota_c).astype(jnp.float32)
    off = lax.dot_general(pc.astype(jnp.float32), sutri,
                          (((1,), (0,)), ((), ())), precision=_HIGHEST,
                          preferred_element_type=jnp.float32).astype(jnp.int32)
    vend = off + c
    pcum = off + pc

    eidx = eidx_ref[...]
    dest = rank_ref[...]
    for e in range(E):
        dest = dest + jnp.where(eidx == e, off[:, e:e + 1], 0)
    dest_ref[...] = dest

    iota8 = lax.broadcasted_iota(jnp.int32, (1, E), 1)
    max_live = jnp.max(jnp.where(c > 0, iota8, 0))
    tt = lax.broadcasted_iota(jnp.int32, (1, T), 1) * M
    be = jnp.zeros((1, T), jnp.int32)
    for e in range(E):
        be = be + (tt >= pcum[:, e:e + 1]).astype(jnp.int32)
    be = jnp.minimum(be, max_live)
    be_ref[...] = be
    vend_t = jnp.zeros((1, T), jnp.int32)
    for e in range(E):
        vend_t = vend_t + jnp.where(be == e, vend[:, e:e + 1], 0)
    tv_ref[...] = jnp.clip(vend_t - tt, 0, M)


def _meta(counts, eidx, rank):
    return pl.pallas_call(
        _meta_body,
        out_shape=[
            jax.ShapeDtypeStruct((N, K), jnp.int32),
            jax.ShapeDtypeStruct((1, T), jnp.int32),
            jax.ShapeDtypeStruct((1, T), jnp.int32),
        ],
    )(counts, eidx, rank)


def _sc_mesh():
    return plsc.VectorSubcoreMesh(core_axis_name="c", subcore_axis_name="s")


def _dispatch_body(x_hbm, dest_hbm, destf_hbm, gatesf_hbm, xd_hbm, gs_hbm,
                   idx_v, idxf_v, gf_v, rows_v, sem_l, sem_s, sem_g):
    wid = lax.axis_index("s") * NC + lax.axis_index("c")
    base = wid * TPW
    pltpu.sync_copy(dest_hbm.at[wid], idx_v)
    pltpu.sync_copy(destf_hbm.at[wid], idxf_v)
    pltpu.sync_copy(gatesf_hbm.at[wid], gf_v)
    gate_sc = [
        pltpu.async_copy(gf_v.at[b], gs_hbm.at[idxf_v.at[b]], sem_g)
        for b in range(K * TPW // 128)
    ]
    loads = {
        j: pltpu.async_copy(x_hbm.at[pl.ds(base + j * CH, CH)],
                            rows_v.at[j % 3], sem_l)
        for j in range(min(2, NJ))
    }
    scat = {}
    waited = set()
    for j in range(NJ):
        loads[j].wait()
        if j + 2 < NJ:
            for h in scat.get(j - 1, ()):
                h.wait()
            waited.add(j - 1)
            loads[j + 2] = pltpu.async_copy(
                x_hbm.at[pl.ds(base + (j + 2) * CH, CH)],
                rows_v.at[(j + 2) % 3], sem_l)
        scat[j] = [
            pltpu.async_copy(rows_v.at[j % 3], xd_hbm.at[idx_v.at[j, k]],
                             sem_s)
            for k in range(K)
        ]
    for j in range(NJ):
        if j not in waited:
            for h in scat[j]:
                h.wait()
    for h in gate_sc:
        h.wait()


def _dispatch_sc(xf, dest3, destf, gatesf):
    return pl.kernel(
        _dispatch_body,
        mesh=_sc_mesh(),
        out_type=[
            jax.ShapeDtypeStruct((P, D), jnp.float32),
            jax.ShapeDtypeStruct((P,), jnp.float32),
        ],
        scratch_types=[
            pltpu.VMEM((NJ, K, CH), jnp.int32),
            pltpu.VMEM((K * TPW // 128, 128), jnp.int32),
            pltpu.VMEM((K * TPW // 128, 128), jnp.float32),
            pltpu.VMEM((3, CH, D), jnp.float32),
            pltpu.SemaphoreType.DMA,
            pltpu.SemaphoreType.DMA,
            pltpu.SemaphoreType.DMA,
        ],
    )(xf, dest3, destf, gatesf)


def _ffn1_body(be_ref, tv_ref, xd_ref, w1_ref, b1_ref, h_ref):
    t = pl.program_id(0)

    @pl.when(tv_ref[t] > 0)
    def _():
        xb = xd_ref[...].astype(jnp.bfloat16)
        h = lax.dot_general(xb, w1_ref[0], (((1,), (0,)), ((), ())),
                            preferred_element_type=jnp.float32)
        h = h + b1_ref[0]
        h = 0.5 * h * (1.0 + lax.erf(h * 0.7071067811865476))
        h_ref[...] = h.astype(jnp.bfloat16)


def _ffn2_body(be_ref, tv_ref, h_ref, w2_ref, b2_ref, gs_ref, out_ref):
    t = pl.program_id(0)

    @pl.when(tv_ref[t] > 0)
    def _():
        out = lax.dot_general(h_ref[...], w2_ref[0], (((1,), (0,)), ((), ())),
                              preferred_element_type=jnp.float32)
        g = gs_ref[0, 0, :]
        out_ref[...] = (out + b2_ref[0]) * g[:, None]


def _ffn(be_arr, tv_arr, xd, w1b, w2b, b1, b2, gslot):
    def live(t, tv):
        return jnp.where(tv[t] > 0, t, 0)

    spec1 = pltpu.PrefetchScalarGridSpec(
        num_scalar_prefetch=2,
        grid=(T,),
        in_specs=[
            pl.BlockSpec((M, D), lambda t, be, tv: (live(t, tv), 0)),
            pl.BlockSpec((1, D, H), lambda t, be, tv: (be[t], 0, 0)),
            pl.BlockSpec((1, 1, H), lambda t, be, tv: (be[t], 0, 0)),
        ],
        out_specs=pl.BlockSpec((M, H), lambda t, be, tv: (t, 0)),
    )
    h = pl.pallas_call(
        _ffn1_body,
        grid_spec=spec1,
        out_shape=jax.ShapeDtypeStruct((P, H), jnp.bfloat16),
    )(be_arr, tv_arr, xd, w1b, b1)
    spec2 = pltpu.PrefetchScalarGridSpec(
        num_scalar_prefetch=2,
        grid=(T,),
        in_specs=[
            pl.BlockSpec((M, H), lambda t, be, tv: (live(t, tv), 0)),
            pl.BlockSpec((1, H, D), lambda t, be, tv: (be[t], 0, 0)),
            pl.BlockSpec((1, 1, D), lambda t, be, tv: (be[t], 0, 0)),
            pl.BlockSpec((1, 1, M), lambda t, be, tv: (live(t, tv), 0, 0)),
        ],
        out_specs=pl.BlockSpec((M, D), lambda t, be, tv: (t, 0)),
    )
    return pl.pallas_call(
        _ffn2_body,
        grid_spec=spec2,
        out_shape=jax.ShapeDtypeStruct((P, D), jnp.float32),
    )(be_arr, tv_arr, h, w2b, b2, gslot)


def _combine_body(out_hbm, dest_hbm, y_hbm, idx_v, r0, r1, sem_g, sem_y):
    wid = lax.axis_index("s") * NC + lax.axis_index("c")
    base = wid * TPW
    pltpu.sync_copy(dest_hbm.at[wid], idx_v)

    def gathers(j, b):
        return [
            pltpu.async_copy(out_hbm.at[idx_v.at[j, 0]], r0.at[b], sem_g),
            pltpu.async_copy(out_hbm.at[idx_v.at[j, 1]], r1.at[b], sem_g),
        ]

    pend = gathers(0, 0)
    ysc = None
    for j in range(NJ2):
        b = j % 2
        for hdl in pend:
            hdl.wait()
        if j + 1 < NJ2:
            if ysc is not None:
                ysc.wait()
                ysc = None
            pend = gathers(j + 1, 1 - b)

        def body_i(i, _):
            def body_c(cc, _):
                a = r0[b, i, pl.ds(cc * 16, 16)]
                bb = r1[b, i, pl.ds(cc * 16, 16)]
                r0[b, i, pl.ds(cc * 16, 16)] = a + bb
                return 0

            lax.fori_loop(0, D // 16, body_c, 0)
            return 0

        lax.fori_loop(0, CH2, body_i, 0)
        if ysc is not None:
            ysc.wait()
        ysc = pltpu.async_copy(r0.at[b], y_hbm.at[pl.ds(base + j * CH2, CH2)],
                               sem_y)
    ysc.wait()


def _combine_sc(out, dest2):
    return pl.kernel(
        _combine_body,
        mesh=_sc_mesh(),
        out_type=jax.ShapeDtypeStruct((N, D), jnp.float32),
        scratch_types=[
            pltpu.VMEM((NJ2, K, CH2), jnp.int32),
            pltpu.VMEM((2, CH2, D), jnp.float32),
            pltpu.VMEM((2, CH2, D), jnp.float32),
            pltpu.SemaphoreType.DMA,
            pltpu.SemaphoreType.DMA,
        ],
    )(out, dest2)


@jax.jit
def kernel(x, Wr, br, W1, b1, W2, b2):
    xf = x.reshape(N, D)
    logits = xf @ Wr + br
    gates, eidx, rank, counts = _router(logits)
    dest, be, tv = _meta(counts, eidx, rank)
    dest3 = dest.reshape(NW, NJ, CH, K).transpose(0, 1, 3, 2)
    gates3 = gates.reshape(NW, NJ, CH, K).transpose(0, 1, 3, 2)
    destf = dest3.reshape(NW, K * TPW // 128, 128)
    gatesf = gates3.reshape(NW, K * TPW // 128, 128)
    dest2 = dest.reshape(NW, NJ2, CH2, K).transpose(0, 1, 3, 2)
    xd, gslot = _dispatch_sc(xf, dest3, destf, gatesf)
    out = _ffn(be.reshape(T), tv.reshape(T), xd,
               W1.astype(jnp.bfloat16), W2.astype(jnp.bfloat16),
               b1.reshape(E, 1, H), b2.reshape(E, 1, D),
               gslot.reshape(T, 1, M))
    y = _combine_sc(out, dest2)
    return y.astype(jnp.float32).reshape(B, S, D)
